# baseline (speedup 1.0000x reference)
import functools

import jax
import jax.numpy as jnp
from jax import lax
from jax.experimental import pallas as pl
from jax.experimental.pallas import tpu as pltpu
from jax.experimental.pallas import tpu_sc as plsc

NC = 2
NS = 16
NW = NC * NS
L = 16

B = 4096 * 50
D = 64
CHUNK = 128
CPW = B // CHUNK // NW
NBUF = 2
GROUPS = CHUNK // L


def _make_gather(num_embeddings):
    mesh = plsc.VectorSubcoreMesh(
        core_axis_name="c", subcore_axis_name="s",
        num_cores=NC, num_subcores=NS)

    @functools.partial(
        pl.kernel,
        out_type=jax.ShapeDtypeStruct((B // 2, 2 * D), jnp.float32),
        mesh=mesh,
        scratch_types=(
            [pltpu.VMEM((CPW, CHUNK), jnp.int32)]
            + [pltpu.VMEM((CHUNK,), jnp.int32) for _ in range(NBUF)]
            + [pltpu.VMEM((CHUNK, 2 * D), jnp.float32) for _ in range(NBUF)]
            + [pltpu.VMEM((CHUNK // 2, 2 * D), jnp.float32)
               for _ in range(NBUF)]
            + [pltpu.SemaphoreType.DMA for _ in range(NBUF)]
            + [pltpu.SemaphoreType.DMA for _ in range(NBUF)]
        ),
        compiler_params=pltpu.CompilerParams(needs_layout_passes=False),
    )
    def gather(idx_hbm, table2_hbm, out2_hbm, idx_v, *scr):
        qb = scr[0:NBUF]
        rows = scr[NBUF:2 * NBUF]
        outb = scr[2 * NBUF:3 * NBUF]
        gsem = scr[3 * NBUF:4 * NBUF]
        osem = scr[4 * NBUF:5 * NBUF]
        wid = lax.axis_index("s") * NC + lax.axis_index("c")
        cbase = wid * CPW

        pltpu.sync_copy(idx_hbm.at[wid], idx_v)

        iota = lax.iota(jnp.int32, L)
        rowhalf = lax.shift_right_logical(iota, 1)
        colpar = (iota & 1) * D

        def fire(g, b):
            @pl.loop(0, GROUPS)
            def _(j):
                t16 = idx_v[g, pl.ds(j * L, L)]
                qb[b][pl.ds(j * L, L)] = lax.shift_right_logical(t16, 1)
            pltpu.async_copy(table2_hbm.at[qb[b]], rows[b], gsem[b])

        def wait_gather(b):
            pltpu.make_async_copy(
                table2_hbm.at[qb[b]], rows[b], gsem[b]).wait()

        def extract(g, b):
            @pl.loop(0, GROUPS)
            def _(j):
                t16 = idx_v[g, pl.ds(j * L, L)]
                srcrow = j * L + iota
                srccol0 = (t16 & 1) * D
                dstrow = j * (L // 2) + rowhalf
                for c in range(D):
                    v = plsc.load_gather(rows[b], [srcrow, srccol0 + c])
                    plsc.store_scatter(outb[b], [dstrow, colpar + c], v)

        def store(g, b):
            pltpu.async_copy(
                outb[b],
                out2_hbm.at[pl.ds((cbase + g) * (CHUNK // 2), CHUNK // 2)],
                osem[b])

        def wait_store(b):
            pltpu.make_async_copy(
                outb[b],
                out2_hbm.at[pl.ds(0, CHUNK // 2)], osem[b]).wait()

        for b in range(NBUF):
            fire(b, b)

        @pl.loop(0, CPW - NBUF, step=NBUF)
        def _(outer):
            for b in range(NBUF):
                g = outer + b
                wait_gather(b)
                @pl.when(outer > 0)
                def _():
                    wait_store(b)
                extract(g, b)
                store(g, b)
                fire(g + NBUF, b)

        for b in range(NBUF):
            g = (CPW - NBUF) + b
            wait_gather(b)
            wait_store(b)
            extract(g, b)
            store(g, b)
        for b in range(NBUF):
            wait_store(b)

    return gather


def kernel(token_ids, embedding_matrix):
    n, s = token_ids.shape
    idx = token_ids.astype(jnp.int32).reshape(NW, CPW, CHUNK)
    table2 = embedding_matrix.reshape(embedding_matrix.shape[0] // 2, 2 * D)
    out2 = _make_gather(embedding_matrix.shape[0])(idx, table2)
    return out2.reshape(n, s, D)


# speedup vs baseline: 1.5719x; 1.5719x over previous
import functools

import jax
import jax.numpy as jnp
from jax import lax
from jax.experimental import pallas as pl
from jax.experimental.pallas import tpu as pltpu
from jax.experimental.pallas import tpu_sc as plsc

NC = 2
NS = 16
NW = NC * NS
L = 16

B = 4096 * 50
D = 64
CHUNK = 128
CPW = B // CHUNK // NW
NBUF = 2


def _make_gather(num_embeddings):
    mesh = plsc.VectorSubcoreMesh(
        core_axis_name="c", subcore_axis_name="s",
        num_cores=NC, num_subcores=NS)

    @functools.partial(
        pl.kernel,
        out_type=jax.ShapeDtypeStruct((B // 2, 2 * D), jnp.float32),
        mesh=mesh,
        scratch_types=(
            [pltpu.VMEM((CPW, CHUNK), jnp.int32)]
            + [pltpu.VMEM((CHUNK, 2 * D), jnp.float32) for _ in range(NBUF)]
            + [pltpu.VMEM((CHUNK // 2, 2 * D), jnp.float32)
               for _ in range(NBUF)]
            + [pltpu.SemaphoreType.DMA for _ in range(NBUF)]
            + [pltpu.SemaphoreType.DMA for _ in range(NBUF)]
        ),
        compiler_params=pltpu.CompilerParams(needs_layout_passes=False),
    )
    def gather(idx_hbm, tpad_hbm, out2_hbm, idx_v, *scr):
        rows = scr[0:NBUF]
        outb = scr[NBUF:2 * NBUF]
        gsem = scr[2 * NBUF:3 * NBUF]
        osem = scr[3 * NBUF:4 * NBUF]
        wid = lax.axis_index("s") * NC + lax.axis_index("c")
        cbase = wid * CPW

        pltpu.sync_copy(idx_hbm.at[wid], idx_v)

        def fire(g, b):
            pltpu.async_copy(tpad_hbm.at[idx_v.at[g]], rows[b], gsem[b])

        def wait_gather(b):
            pltpu.make_async_copy(
                tpad_hbm.at[idx_v.at[0]], rows[b], gsem[b]).wait()

        def extract(b):
            @pl.loop(0, CHUNK // 2)
            def _(r):
                for k in range(D // L):
                    outb[b][r, pl.ds(k * L, L)] = \
                        rows[b][2 * r, pl.ds(k * L, L)]
                    outb[b][r, pl.ds(D + k * L, L)] = \
                        rows[b][2 * r + 1, pl.ds(k * L, L)]

        def store(g, b):
            pltpu.async_copy(
                outb[b],
                out2_hbm.at[pl.ds((cbase + g) * (CHUNK // 2), CHUNK // 2)],
                osem[b])

        def wait_store(b):
            pltpu.make_async_copy(
                outb[b],
                out2_hbm.at[pl.ds(0, CHUNK // 2)], osem[b]).wait()

        for b in range(NBUF):
            fire(b, b)

        @pl.loop(0, CPW - NBUF, step=NBUF)
        def _(outer):
            for b in range(NBUF):
                g = outer + b
                wait_gather(b)
                @pl.when(outer > 0)
                def _():
                    wait_store(b)
                extract(b)
                store(g, b)
                fire(g + NBUF, b)

        for b in range(NBUF):
            g = (CPW - NBUF) + b
            wait_gather(b)
            wait_store(b)
            extract(b)
            store(g, b)
        for b in range(NBUF):
            wait_store(b)

    return gather


def kernel(token_ids, embedding_matrix):
    n, s = token_ids.shape
    idx = token_ids.astype(jnp.int32).reshape(NW, CPW, CHUNK)
    tpad = jnp.pad(embedding_matrix, ((0, 0), (0, D)))
    out2 = _make_gather(embedding_matrix.shape[0])(idx, tpad)
    return out2.reshape(n, s, D)
